# speedup vs baseline: 5.8086x; 5.8086x over previous
import functools

import jax
import jax.numpy as jnp
from jax import lax
from jax.experimental import pallas as pl
from jax.experimental.pallas import tpu as pltpu
from jax.experimental.pallas import tpu_sc as plsc

INPUT_DIM = 10000
D = 128
NC = 2
NS = 16
NW = NC * NS
B = 4096 * 200
B_PER_W = B // NW
CHUNK = 128
N_CHUNKS = B_PER_W // CHUNK

_mesh = plsc.VectorSubcoreMesh(core_axis_name="c", subcore_axis_name="s")


@functools.partial(
    pl.kernel,
    mesh=_mesh,
    out_type=jax.ShapeDtypeStruct((B, D), jnp.float32),
    scratch_types=[
        pltpu.VMEM((CHUNK,), jnp.int32),
        pltpu.VMEM((CHUNK, D), jnp.float32),
        pltpu.SemaphoreType.DMA,
    ],
)
def _embed_lookup(idx_hbm, table_hbm, out_hbm, idx_v, rows_v, sem):
    wid = lax.axis_index("s") * NC + lax.axis_index("c")
    base = wid * B_PER_W

    def body(i, carry):
        off = base + i * CHUNK
        pltpu.sync_copy(idx_hbm.at[pl.ds(off, CHUNK)], idx_v)
        pltpu.async_copy(table_hbm.at[idx_v], rows_v, sem).wait()
        pltpu.sync_copy(rows_v, out_hbm.at[pl.ds(off, CHUNK)])
        return carry

    lax.fori_loop(0, N_CHUNKS, body, 0)


def kernel(x, table):
    idx = x.reshape(-1).astype(jnp.int32)
    out = _embed_lookup(idx, table)
    return out.reshape(x.shape + (D,))


# speedup vs baseline: 9.5285x; 1.6404x over previous
import functools

import jax
import jax.numpy as jnp
from jax import lax
from jax.experimental import pallas as pl
from jax.experimental.pallas import tpu as pltpu
from jax.experimental.pallas import tpu_sc as plsc

D = 128
NC = 2
NS = 16
NW = NC * NS
B = 4096 * 200
B_PER_W = B // NW
CHUNK = 128
SUP = 8
N_SUP = B_PER_W // (SUP * CHUNK)
ROWS_PER_W = B_PER_W // CHUNK

_mesh = plsc.VectorSubcoreMesh(core_axis_name="c", subcore_axis_name="s")


@functools.partial(
    pl.kernel,
    mesh=_mesh,
    out_type=jax.ShapeDtypeStruct((B, D), jnp.float32),
    scratch_types=[
        pltpu.VMEM((SUP, CHUNK), jnp.int32),
        pltpu.VMEM((2, CHUNK, D), jnp.float32),
        pltpu.SemaphoreType.DMA,
        pltpu.SemaphoreType.DMA,
    ],
)
def _embed_lookup(idx_hbm, table_hbm, out_hbm, idx_v, rows_v, gsem, wsem):
    wid = lax.axis_index("s") * NC + lax.axis_index("c")
    base = wid * B_PER_W
    irow = wid * ROWS_PER_W

    def body(s, carry):
        pltpu.sync_copy(idx_hbm.at[pl.ds(irow + s * SUP, SUP)], idx_v)
        obase = base + s * SUP * CHUNK

        def gat(j):
            return pltpu.async_copy(table_hbm.at[idx_v.at[j]], rows_v.at[j % 2], gsem)

        def wrt(j):
            return pltpu.async_copy(
                rows_v.at[j % 2], out_hbm.at[pl.ds(obase + j * CHUNK, CHUNK)], wsem)

        g = [None] * SUP
        w = [None] * SUP
        g[0] = gat(0)
        for j in range(1, SUP):
            if j >= 2:
                w[j - 2].wait()
            g[j] = gat(j)
            g[j - 1].wait()
            w[j - 1] = wrt(j - 1)
        g[SUP - 1].wait()
        w[SUP - 1] = wrt(SUP - 1)
        w[SUP - 2].wait()
        w[SUP - 1].wait()
        return carry

    lax.fori_loop(0, N_SUP, body, 0)


def kernel(x, table):
    idx = x.reshape(-1, CHUNK).astype(jnp.int32)
    out = _embed_lookup(idx, table)
    return out.reshape(x.shape + (D,))


# speedup vs baseline: 10.0160x; 1.0512x over previous
import functools

import jax
import jax.numpy as jnp
from jax import lax
from jax.experimental import pallas as pl
from jax.experimental.pallas import tpu as pltpu
from jax.experimental.pallas import tpu_sc as plsc

D = 128
NC = 2
NS = 16
NW = NC * NS
B = 4096 * 200
B_PER_W = B // NW
CHUNK = 128
U = 20
N_SUP = B_PER_W // (U * CHUNK)
NBUF = 4

_mesh = plsc.VectorSubcoreMesh(core_axis_name="c", subcore_axis_name="s")


@functools.partial(
    pl.kernel,
    mesh=_mesh,
    out_type=jax.ShapeDtypeStruct((B, D), jnp.float32),
    scratch_types=[
        pltpu.VMEM((N_SUP, U, CHUNK), jnp.int32),
        pltpu.VMEM((NBUF * CHUNK, D), jnp.float32),
        pltpu.SemaphoreType.DMA,
        pltpu.SemaphoreType.DMA,
    ],
)
def _embed_lookup(idx_hbm, table_hbm, out_hbm, idx_v, rows_v, gsem, wsem):
    wid = lax.axis_index("s") * NC + lax.axis_index("c")
    base = wid * B_PER_W

    pltpu.sync_copy(idx_hbm.at[wid], idx_v)

    def body(s, carry):
        obase = base + s * U * CHUNK

        def gat(j):
            return pltpu.async_copy(
                table_hbm.at[idx_v.at[s, j]],
                rows_v.at[pl.ds((j % NBUF) * CHUNK, CHUNK)], gsem)

        def wrt(j):
            return pltpu.async_copy(
                rows_v.at[pl.ds((j % NBUF) * CHUNK, CHUNK)],
                out_hbm.at[pl.ds(obase + j * CHUNK, CHUNK)], wsem)

        g = [None] * U
        w = [None] * U
        for j in range(U):
            if j >= NBUF:
                w[j - NBUF].wait()
            g[j] = gat(j)
            if j >= 1:
                g[j - 1].wait()
                w[j - 1] = wrt(j - 1)
        g[U - 1].wait()
        w[U - 1] = wrt(U - 1)
        for k in range(U - NBUF, U):
            w[k].wait()
        return carry

    lax.fori_loop(0, N_SUP, body, 0)


def kernel(x, table):
    idx = x.reshape(NW, N_SUP, U, CHUNK).astype(jnp.int32)
    out = _embed_lookup(idx, table)
    return out.reshape(x.shape + (D,))


# speedup vs baseline: 16.5142x; 1.6488x over previous
import functools

import jax
import jax.numpy as jnp
from jax import lax
from jax.experimental import pallas as pl
from jax.experimental.pallas import tpu as pltpu
from jax.experimental.pallas import tpu_sc as plsc

V = 10000
D = 128
NC = 2
NS = 16
NW = NC * NS
B = 4096 * 200
B_PER_W = B // NW
CHUNK = 128
U = 8
N_SUP = B_PER_W // (U * CHUNK)
NBUF = 2
FILL_T = 10

_mesh = plsc.VectorSubcoreMesh(core_axis_name="c", subcore_axis_name="s")


@functools.partial(
    pl.kernel,
    mesh=_mesh,
    out_type=jax.ShapeDtypeStruct((B, D), jnp.float32),
    scratch_types=[
        pltpu.VMEM_SHARED((V, D), jnp.float32),
        pltpu.VMEM((2, U, CHUNK), jnp.int32),
        pltpu.VMEM((NBUF * CHUNK, D), jnp.float32),
        pltpu.SemaphoreType.DMA,
        pltpu.SemaphoreType.DMA,
        pltpu.SemaphoreType.DMA,
    ],
)
def _embed_lookup(idx_hbm, table_hbm, out_hbm, tab_sh, idx_v, rows_v,
                  gsem, wsem, isem):
    sid = lax.axis_index("s")
    wid = sid * NC + lax.axis_index("c")
    base = wid * B_PER_W

    rows_per_fill = V // FILL_T

    @pl.when(sid < FILL_T)
    def _fill():
        pltpu.sync_copy(
            table_hbm.at[pl.ds(sid * rows_per_fill, rows_per_fill)],
            tab_sh.at[pl.ds(sid * rows_per_fill, rows_per_fill)])

    pltpu.async_copy(idx_hbm.at[wid, 0], idx_v.at[0], isem)

    plsc.subcore_barrier()

    def body(s, carry):
        pb = lax.rem(s, 2)
        obase = base + s * U * CHUNK

        pltpu.make_async_copy(idx_hbm.at[wid, s], idx_v.at[pb], isem).wait()

        @pl.when(s + 1 < N_SUP)
        def _prefetch():
            pltpu.async_copy(idx_hbm.at[wid, s + 1], idx_v.at[1 - pb], isem)

        def gat(j):
            return pltpu.async_copy(
                tab_sh.at[idx_v.at[pb, j]],
                rows_v.at[pl.ds((j % NBUF) * CHUNK, CHUNK)], gsem)

        def wrt(j):
            return pltpu.async_copy(
                rows_v.at[pl.ds((j % NBUF) * CHUNK, CHUNK)],
                out_hbm.at[pl.ds(obase + j * CHUNK, CHUNK)], wsem)

        g = [None] * U
        w = [None] * U
        for j in range(U):
            if j >= NBUF:
                w[j - NBUF].wait()
            g[j] = gat(j)
            if j >= 1:
                g[j - 1].wait()
                w[j - 1] = wrt(j - 1)
        g[U - 1].wait()
        w[U - 1] = wrt(U - 1)
        for k in range(U - NBUF, U):
            w[k].wait()
        return carry

    lax.fori_loop(0, N_SUP, body, 0)


def kernel(x, table):
    idx = x.reshape(NW, N_SUP, U, CHUNK).astype(jnp.int32)
    out = _embed_lookup(idx, table)
    return out.reshape(x.shape + (D,))


# speedup vs baseline: 17.4307x; 1.0555x over previous
import functools

import jax
import jax.numpy as jnp
from jax import lax
from jax.experimental import pallas as pl
from jax.experimental.pallas import tpu as pltpu
from jax.experimental.pallas import tpu_sc as plsc

V = 10000
D = 128
NC = 2
NS = 16
NW = NC * NS
B = 4096 * 200
B_PER_W = B // NW
CHUNK = 128
U = 8
N_SUP = B_PER_W // (U * CHUNK)
NBUF = 2
FILL_T = 10

_mesh = plsc.VectorSubcoreMesh(core_axis_name="c", subcore_axis_name="s")


@functools.partial(
    pl.kernel,
    mesh=_mesh,
    out_type=jax.ShapeDtypeStruct((B, D), jnp.float32),
    scratch_types=[
        pltpu.VMEM_SHARED((V, D), jnp.float32),
        pltpu.VMEM((2, U, CHUNK), jnp.int32),
        pltpu.VMEM((NBUF * CHUNK, D), jnp.float32),
        pltpu.SemaphoreType.DMA,
        pltpu.SemaphoreType.DMA,
        pltpu.SemaphoreType.DMA,
    ],
)
def _embed_lookup(idx_hbm, table_hbm, out_hbm, tab_sh, idx_v, rows_v,
                  gsem, wsem, isem):
    sid = lax.axis_index("s")
    wid = sid * NC + lax.axis_index("c")
    base = wid * B_PER_W

    rows_per_fill = V // FILL_T

    @pl.when(sid < FILL_T)
    def _fill():
        pltpu.sync_copy(
            table_hbm.at[pl.ds(sid * rows_per_fill, rows_per_fill)],
            tab_sh.at[pl.ds(sid * rows_per_fill, rows_per_fill)])

    pltpu.async_copy(idx_hbm.at[wid, 0], idx_v.at[0], isem)

    plsc.subcore_barrier()

    def buf(j):
        return rows_v.at[pl.ds((j % NBUF) * CHUNK, CHUNK)]

    def wait_one_write(wsem_):
        pltpu.make_async_copy(buf(0), out_hbm.at[pl.ds(base, CHUNK)], wsem_).wait()

    def wait_one_gather(gsem_):
        pltpu.make_async_copy(
            tab_sh.at[idx_v.at[0, 0]], buf(0), gsem_).wait()

    def body(s, carry):
        pb = lax.rem(s, 2)
        obase = base + s * U * CHUNK

        pltpu.make_async_copy(idx_hbm.at[wid, s], idx_v.at[pb], isem).wait()

        @pl.when(s + 1 < N_SUP)
        def _prefetch():
            pltpu.async_copy(idx_hbm.at[wid, s + 1], idx_v.at[1 - pb], isem)

        def gat(j):
            return pltpu.async_copy(tab_sh.at[idx_v.at[pb, j]], buf(j), gsem)

        def wrt(j):
            return pltpu.async_copy(
                buf(j), out_hbm.at[pl.ds(obase + j * CHUNK, CHUNK)], wsem)

        g = [None] * U
        w = [None] * U
        for j in range(U):
            if j >= NBUF:
                w[j - NBUF].wait()
            else:
                @pl.when(s > 0)
                def _wfree():
                    wait_one_write(wsem)
            g[j] = gat(j)
            if j >= 1:
                g[j - 1].wait()
                w[j - 1] = wrt(j - 1)
            else:
                @pl.when(s > 0)
                def _wlast():
                    wait_one_gather(gsem)
                    pltpu.async_copy(
                        buf(U - 1),
                        out_hbm.at[pl.ds(obase - CHUNK, CHUNK)], wsem)
        return carry

    lax.fori_loop(0, N_SUP, body, 0)

    wait_one_gather(gsem)
    pltpu.async_copy(
        buf(U - 1), out_hbm.at[pl.ds(base + B_PER_W - CHUNK, CHUNK)], wsem)
    wait_one_write(wsem)
    wait_one_write(wsem)


def kernel(x, table):
    idx = x.reshape(NW, N_SUP, U, CHUNK).astype(jnp.int32)
    out = _embed_lookup(idx, table)
    return out.reshape(x.shape + (D,))
